# Initial kernel scaffold; baseline (speedup 1.0000x reference)
import jax
import jax.numpy as jnp
from jax.experimental import pallas as pl


def kernel(x, W_emb, b_emb, ln1_s, ln1_b, Wqk, Wv, Wo, ln2_s, ln2_b, Wff1, bff1, Wff2, bff2, lnf_s, lnf_b, Wf1, bf1, Wf2, bf2):
    raise NotImplementedError("write your pallas kernel here")


# speedup vs baseline: 2.3881x; 2.3881x over previous
import functools
import jax
import jax.numpy as jnp
from jax.experimental import pallas as pl
from jax.experimental.pallas import tpu as pltpu

_B = 2
_S = 2048
_D = 768
_H = 12
_DH = 64
_NH = 2
_BK = 64
_NB = _S // _BK
_NC = _NH * _NB
_ROWS = _B * _S
_RB = 512
_CH = 256
_G = 4
_GR = _G * _BK
_KR = (_G + 1) * _BK

_f32 = jnp.float32
_PH = jax.lax.Precision.HIGHEST


def _fiota(shape, dim):
    return jax.lax.broadcasted_iota(jnp.int32, shape, dim).astype(_f32)

def _ln(x, s, b):
    mu = jnp.mean(x, axis=-1, keepdims=True)
    xc = x - mu
    var = jnp.mean(xc * xc, axis=-1, keepdims=True)
    return xc * jax.lax.rsqrt(var + 1e-5) * s + b


def _embed_body(x_ref, w_ref, b_ref, o_ref):
    o_ref[...] = jnp.dot(x_ref[...], w_ref[...],
                         preferred_element_type=_f32) + b_ref[...]


def _ln_qkv_body(h_ref, s_ref, b_ref, wqk_ref, wv_ref, qk_ref, v_ref):
    y = _ln(h_ref[...], s_ref[...], b_ref[...])
    qk_ref[...] = jnp.dot(y, wqk_ref[...], preferred_element_type=_f32)
    v_ref[...] = jnp.dot(y, wv_ref[...], preferred_element_type=_f32)


def _resid_wo_body(h_ref, a_ref, wo_ref, o_ref):
    o_ref[...] = h_ref[...] + jnp.dot(a_ref[...], wo_ref[...],
                                      preferred_element_type=_f32)


def _ff_body(h_ref, s_ref, b_ref, w1_ref, b1_ref, w2_ref, b2_ref, o_ref):
    h = h_ref[...]
    y = _ln(h, s_ref[...], b_ref[...])
    u = jnp.maximum(jnp.dot(y, w1_ref[...], preferred_element_type=_f32)
                    + b1_ref[...], 0.0)
    o_ref[...] = h + jnp.dot(u, w2_ref[...],
                             preferred_element_type=_f32) + b2_ref[...]


def _head_body(x_ref, s_ref, b_ref, w1_ref, b1_ref, w2_ref, b2_ref, o_ref):
    y = _ln(x_ref[...], s_ref[...], b_ref[...])
    u = jnp.maximum(jnp.dot(y, w1_ref[...], preferred_element_type=_f32)
                    + b1_ref[...], 0.0)
    o_ref[...] = jnp.dot(u, w2_ref[...],
                         preferred_element_type=_f32) + b2_ref[...]


def _attn_body(qk_ref, v_ref, bk_ref, o_ref):
    qk = qk_ref[0, 0]
    v = v_ref[0, 0]
    bks = bk_ref[0, 0]

    pos_col = _fiota((_S, 1), 0)

    ii = _fiota((128, 128), 0)
    jj = _fiota((128, 128), 1)
    tril = (jj < ii).astype(_f32)
    bi = _fiota((_NB, _NB), 0)
    bj = _fiota((_NB, _NB), 1)
    ustri = (bi < bj).astype(_f32)

    lane_nb = _fiota((1, _NB), 1)

    sorted_parts = []
    dests = []
    for r in range(_NH):
        bucket = bks[:, r:r + 1]
        onehot = (bucket == lane_nb).astype(_f32)
        hist = jnp.sum(onehot, axis=0, keepdims=True)
        offs = jnp.dot(hist, ustri, preferred_element_type=_f32, precision=_PH)

        carry = jnp.zeros((1, _NB), _f32)
        dest_blocks = []
        for blk in range(_S // 128):
            rows = onehot[blk * 128:(blk + 1) * 128]
            pref = jnp.dot(tril, rows, preferred_element_type=_f32, precision=_PH) + carry
            dest_blk = jnp.sum((pref + offs) * rows, axis=1, keepdims=True)
            dest_blocks.append(dest_blk)
            carry = carry + jnp.sum(rows, axis=0, keepdims=True)
        dest = jnp.concatenate(dest_blocks, axis=0)
        dests.append(dest)

        g = jnp.concatenate([qk, v, pos_col], axis=1)
        schunks = []
        for c0 in range(0, _S, _CH):
            lane_s = _fiota((1, _CH), 1) + float(c0)
            pt = (dest == lane_s).astype(_f32)
            schunks.append(jax.lax.dot_general(
                pt, g, (((0,), (0,)), ((), ())),
                preferred_element_type=_f32, precision=_PH))
        sorted_parts.append(jnp.concatenate(schunks, axis=0))

    sall = jnp.concatenate(sorted_parts, axis=0)
    sqk = sall[:, :_DH]
    sv = sall[:, _DH:2 * _DH]
    spos = sall[:, 2 * _DH:2 * _DH + 1]
    nrm = jnp.sqrt(jnp.sum(sqk * sqk, axis=1, keepdims=True))
    sk = sqk / (nrm + 1e-6)

    ei = _fiota((_KR, _KR), 0)
    ej = _fiota((_KR, _KR), 1)
    eye = (ei == ej).astype(_f32)

    n_groups = _NC // _G
    so_parts = []
    for gidx in range(n_groups):
        q0 = gidx * _GR
        q = sqk[q0:q0 + _GR]
        pq = spos[q0:q0 + _GR]
        if gidx == 0:
            kwin = jnp.concatenate([sk[_NH * _S - _BK:], sk[:_GR]], axis=0)
            vwin = jnp.concatenate([sv[_NH * _S - _BK:], sv[:_GR]], axis=0)
            pwin = jnp.concatenate([spos[_NH * _S - _BK:], spos[:_GR]],
                                   axis=0)
        else:
            kwin = sk[q0 - _BK:q0 + _GR]
            vwin = sv[q0 - _BK:q0 + _GR]
            pwin = spos[q0 - _BK:q0 + _GR]
        pk = jax.lax.dot_general(pwin, eye, (((0,), (0,)), ((), ())),
                                 preferred_element_type=_f32, precision=_PH)

        dots = jax.lax.dot_general(q, kwin, (((1,), (1,)), ((), ())),
                                   preferred_element_type=_f32) * (_DH ** -0.5)
        qc = _fiota((_GR, 1), 0) // float(_BK)
        kc = _fiota((1, _KR), 1) // float(_BK) - 1.0
        in_win = jnp.logical_or(kc == qc, kc == qc - 1.0)
        dots = jnp.where(pq < pk, -1e9, dots)
        dots = jnp.where(pq == pk, -1e5, dots)
        dots = jnp.where(in_win, dots, -1e9)
        mx = jnp.max(dots, axis=1, keepdims=True)
        p = jnp.exp(dots - mx)
        ssum = jnp.sum(p, axis=1, keepdims=True)
        logit = mx + jnp.log(ssum)
        o = jax.lax.dot_general(p / ssum, vwin, (((1,), (0,)), ((), ())),
                                preferred_element_type=_f32)
        so_parts.append(jnp.concatenate([o, logit], axis=1))
    so = jnp.concatenate(so_parts, axis=0)

    outs = []
    for r in range(_NH):
        so_r = so[r * _S:(r + 1) * _S]
        dest = dests[r]
        chunks = []
        for i0 in range(0, _S, _CH):
            lane_s = _fiota((1, _S), 1)
            u = (dest[i0:i0 + _CH] == lane_s).astype(_f32)
            chunks.append(jnp.dot(u, so_r, preferred_element_type=_f32, precision=_PH))
        outs.append(jnp.concatenate(chunks, axis=0))

    lg0 = outs[0][:, _DH:_DH + 1]
    lg1 = outs[1][:, _DH:_DH + 1]
    mx = jnp.maximum(lg0, lg1)
    lse = mx + jnp.log(jnp.exp(lg0 - mx) + jnp.exp(lg1 - mx))
    o = (outs[0][:, :_DH] * jnp.exp(lg0 - lse)
         + outs[1][:, :_DH] * jnp.exp(lg1 - lse))
    o_ref[0, 0] = o


def _row_grid_call(body, ins, n_out=1, out_cols=_D, extra_full=()):
    specs = []
    args = []
    for a, blocked in ins:
        args.append(a)
        if blocked:
            specs.append(pl.BlockSpec((_RB, a.shape[1]), lambda i: (i, 0)))
        else:
            specs.append(pl.BlockSpec(a.shape,
                                      lambda i, nd=a.ndim: (0,) * nd))
    out_shape = [jax.ShapeDtypeStruct((_ROWS, out_cols), _f32)
                 for _ in range(n_out)]
    out_specs = [pl.BlockSpec((_RB, out_cols), lambda i: (i, 0))
                 for _ in range(n_out)]
    if n_out == 1:
        out_shape, out_specs = out_shape[0], out_specs[0]
    return pl.pallas_call(
        body,
        grid=(_ROWS // _RB,),
        in_specs=specs,
        out_specs=out_specs,
        out_shape=out_shape,
        compiler_params=pltpu.CompilerParams(
            dimension_semantics=("parallel",)),
    )(*args)


def _attention(qkh, vh, bks):
    return pl.pallas_call(
        _attn_body,
        grid=(_B, _H),
        in_specs=[
            pl.BlockSpec((1, 1, _S, _DH), lambda b, h: (b, h, 0, 0)),
            pl.BlockSpec((1, 1, _S, _DH), lambda b, h: (b, h, 0, 0)),
            pl.BlockSpec((1, 1, _S, _NH), lambda b, h: (b, h, 0, 0)),
        ],
        out_specs=pl.BlockSpec((1, 1, _S, _DH), lambda b, h: (b, h, 0, 0)),
        out_shape=jax.ShapeDtypeStruct((_B, _H, _S, _DH), _f32),
        compiler_params=pltpu.CompilerParams(
            dimension_semantics=("parallel", "parallel")),
    )(qkh, vh, bks)


def kernel(x, W_emb, b_emb, ln1_s, ln1_b, Wqk, Wv, Wo, ln2_s, ln2_b,
           Wff1, bff1, Wff2, bff2, lnf_s, lnf_b, Wf1, bf1, Wf2, bf2):
    rot3 = jax.random.normal(jax.random.key(42), (_DH, _NH, _NB // 2),
                             dtype=_f32)
    x2 = x.reshape(_ROWS, x.shape[-1])

    def _bucket_ids(h2, s, b, wqk):
        h3 = h2.reshape(_B, _S, _D)
        mu = jnp.mean(h3, axis=-1, keepdims=True)
        var = jnp.var(h3, axis=-1, keepdims=True)
        y = (h3 - mu) / jnp.sqrt(var + 1e-5) * s + b
        qk = (y @ wqk).reshape(_B, _S, _H, _DH).transpose(0, 2, 1, 3)
        rotated = jnp.einsum('bhsd,dnr->bhnsr', qk, rot3)
        bk = jnp.argmax(jnp.concatenate([rotated, -rotated], axis=-1),
                        axis=-1)
        return bk.transpose(0, 1, 3, 2).astype(_f32)

    h = _row_grid_call(_embed_body,
                       [(x2, True), (W_emb, False),
                        (b_emb.reshape(1, _D), False)])
    for l in range(Wqk.shape[0]):
        qkh, vh = _row_grid_call(
            _ln_qkv_body,
            [(h, True), (ln1_s[l].reshape(1, _D), False),
             (ln1_b[l].reshape(1, _D), False),
             (Wqk[l], False), (Wv[l], False)],
            n_out=2)
        qkh_t = qkh.reshape(_B, _S, _H, _DH).transpose(0, 2, 1, 3)
        vh_t = vh.reshape(_B, _S, _H, _DH).transpose(0, 2, 1, 3)
        bks = _bucket_ids(h, ln1_s[l], ln1_b[l], Wqk[l])
        a = _attention(qkh_t, vh_t, bks)
        a2 = a.transpose(0, 2, 1, 3).reshape(_ROWS, _D)
        h = _row_grid_call(
            _resid_wo_body,
            [(h, True), (a2, True), (Wo[l], False)])
        h = _row_grid_call(
            _ff_body,
            [(h, True), (ln2_s[l].reshape(1, _D), False),
             (ln2_b[l].reshape(1, _D), False),
             (Wff1[l], False), (bff1[l].reshape(1, 4 * _D), False),
             (Wff2[l], False), (bff2[l].reshape(1, _D), False)])

    last = h.reshape(_B, _S, _D)[:, -1, :]
    out = pl.pallas_call(
        _head_body,
        out_shape=jax.ShapeDtypeStruct((_B, Wf2.shape[1]), _f32),
    )(last, lnf_s.reshape(1, _D), lnf_b.reshape(1, _D),
      Wf1, bf1.reshape(1, -1), Wf2, bf2.reshape(1, -1))
    return out


# speedup vs baseline: 5.7105x; 2.3913x over previous
import functools
import jax
import jax.numpy as jnp
from jax import lax
from jax.experimental import pallas as pl
from jax.experimental.pallas import tpu as pltpu
from jax.experimental.pallas import tpu_sc as plsc

_B = 2
_S = 2048
_D = 768
_H = 12
_DH = 64
_NH = 2
_BK = 64
_NB = _S // _BK
_NC = _NH * _NB
_ROWS = _B * _S
_RB = 512
_G = 4
_GR = _G * _BK
_KR = (_G + 1) * _BK
_N1 = _B * _H * _S
_N2 = _B * _H * _NH * _S
_TW = 256
_OW = 128
_NWORK = 32
_SCH = 128

_f32 = jnp.float32
_PH = jax.lax.Precision.HIGHEST


def _fiota(shape, dim):
    return lax.broadcasted_iota(jnp.int32, shape, dim).astype(_f32)


def _ln(x, s, b):
    mu = jnp.mean(x, axis=-1, keepdims=True)
    xc = x - mu
    var = jnp.mean(xc * xc, axis=-1, keepdims=True)
    return xc / jnp.sqrt(var + 1e-5) * s + b


def _embed_body(x_ref, w_ref, b_ref, o_ref):
    o_ref[...] = jnp.dot(x_ref[...], w_ref[...],
                         preferred_element_type=_f32) + b_ref[...]


def _ln_qkv_body(h_ref, s_ref, b_ref, wqk_ref, wv_ref, qk_ref, v_ref):
    y = _ln(h_ref[...], s_ref[...], b_ref[...])
    qk_ref[...] = jnp.dot(y, wqk_ref[...], preferred_element_type=_f32)
    v_ref[...] = jnp.dot(y, wv_ref[...], preferred_element_type=_f32)


def _resid_wo_body(h_ref, a_ref, wo_ref, o_ref):
    o_ref[...] = h_ref[...] + jnp.dot(a_ref[...], wo_ref[...],
                                      preferred_element_type=_f32)


def _ff_body(h_ref, s_ref, b_ref, w1_ref, b1_ref, w2_ref, b2_ref, o_ref):
    h = h_ref[...]
    y = _ln(h, s_ref[...], b_ref[...])
    u = jnp.maximum(jnp.dot(y, w1_ref[...], preferred_element_type=_f32)
                    + b1_ref[...], 0.0)
    o_ref[...] = h + jnp.dot(u, w2_ref[...],
                             preferred_element_type=_f32) + b2_ref[...]


def _head_body(x_ref, s_ref, b_ref, w1_ref, b1_ref, w2_ref, b2_ref, o_ref):
    y = _ln(x_ref[...], s_ref[...], b_ref[...])
    u = jnp.maximum(jnp.dot(y, w1_ref[...], preferred_element_type=_f32)
                    + b1_ref[...], 0.0)
    o_ref[...] = jnp.dot(u, w2_ref[...],
                         preferred_element_type=_f32) + b2_ref[...]


def _row_grid_call(body, ins, n_out=1, out_cols=_D):
    specs = []
    args = []
    for a, blocked in ins:
        args.append(a)
        if blocked:
            specs.append(pl.BlockSpec((_RB, a.shape[1]), lambda i: (i, 0)))
        else:
            specs.append(pl.BlockSpec(a.shape,
                                      lambda i, nd=a.ndim: (0,) * nd))
    out_shape = [jax.ShapeDtypeStruct((_ROWS, out_cols), _f32)
                 for _ in range(n_out)]
    out_specs = [pl.BlockSpec((_RB, out_cols), lambda i: (i, 0))
                 for _ in range(n_out)]
    if n_out == 1:
        out_shape, out_specs = out_shape[0], out_specs[0]
    return pl.pallas_call(
        body,
        grid=(_ROWS // _RB,),
        in_specs=specs,
        out_specs=out_specs,
        out_shape=out_shape,
        compiler_params=pltpu.CompilerParams(
            dimension_semantics=("parallel",)),
    )(*args)


def _prep_body(bk_ref, dest_ref):
    bks = bk_ref[0, 0]

    ii = _fiota((128, 128), 0)
    jj = _fiota((128, 128), 1)
    tril = (jj < ii).astype(_f32)
    bi = _fiota((_NB, _NB), 0)
    bj = _fiota((_NB, _NB), 1)
    ustri = (bi < bj).astype(_f32)
    lane_nb = _fiota((1, _NB), 1)

    for r in range(_NH):
        bucket = bks[:, r:r + 1]
        onehot = (bucket == lane_nb).astype(_f32)
        hist = jnp.sum(onehot, axis=0, keepdims=True)
        offs = jnp.dot(hist, ustri,
                       preferred_element_type=_f32, precision=_PH)
        carry = jnp.zeros((1, _NB), _f32)
        dest_blocks = []
        for blk in range(_S // 128):
            rows = onehot[blk * 128:(blk + 1) * 128]
            pref = jnp.dot(tril, rows,
                           preferred_element_type=_f32, precision=_PH) + carry
            dest_blocks.append(
                jnp.sum((pref + offs) * rows, axis=1, keepdims=True))
            carry = carry + jnp.sum(rows, axis=0, keepdims=True)
        dest_ref[0, 0, :, r:r + 1] = jnp.concatenate(dest_blocks, axis=0)


def _prep(bks):
    return pl.pallas_call(
        _prep_body,
        grid=(_B, _H),
        in_specs=[pl.BlockSpec((1, 1, _S, _NH), lambda b, h: (b, h, 0, 0))],
        out_specs=pl.BlockSpec((1, 1, _S, _NH), lambda b, h: (b, h, 0, 0)),
        out_shape=jax.ShapeDtypeStruct((_B, _H, _S, _NH), _f32),
        compiler_params=pltpu.CompilerParams(
            dimension_semantics=("parallel", "parallel")),
    )(bks)


def _sc_scatter(table, idx, out_rows, rounds):
    n, w = table.shape
    per_w = n // _NWORK
    n_ch = per_w // _SCH
    mesh = plsc.VectorSubcoreMesh(core_axis_name="c", subcore_axis_name="s")

    @functools.partial(
        pl.kernel, mesh=mesh,
        out_type=jax.ShapeDtypeStruct((out_rows, w), _f32),
        scratch_types=[
            pltpu.VMEM((_SCH,), jnp.int32),
            pltpu.VMEM((_SCH, w), _f32),
            pltpu.SemaphoreType.DMA,
        ],
    )
    def k(table_hbm, idx_hbm, out_hbm, idx_v, rows_v, sem):
        wid = lax.axis_index("s") * 2 + lax.axis_index("c")
        for ch in range(n_ch):
            base = wid * per_w + ch * _SCH
            pltpu.sync_copy(table_hbm.at[pl.ds(base, _SCH)], rows_v)
            for r in range(rounds):
                pltpu.sync_copy(idx_hbm.at[pl.ds(r * n + base, _SCH)], idx_v)
                pltpu.async_copy(rows_v, out_hbm.at[idx_v], sem).wait()

    return k(table, idx)


def _attn_body(st_ref, o_ref):
    blk = st_ref[0, 0]
    sqk = blk[:, :_DH]
    sv = blk[:, _DH:2 * _DH]
    spos = blk[:, 2 * _DH:2 * _DH + 1]
    nrm = jnp.sqrt(jnp.sum(sqk * sqk, axis=1, keepdims=True))
    sk = sqk / (nrm + 1e-6)

    ei = _fiota((_KR, _KR), 0)
    ej = _fiota((_KR, _KR), 1)
    eye = (ei == ej).astype(_f32)

    n_groups = _NC // _G
    so_parts = []
    for gidx in range(n_groups):
        q0 = gidx * _GR
        q = sqk[q0:q0 + _GR]
        pq = spos[q0:q0 + _GR]
        if gidx == 0:
            kwin = jnp.concatenate([sk[_NH * _S - _BK:], sk[:_GR]], axis=0)
            vwin = jnp.concatenate([sv[_NH * _S - _BK:], sv[:_GR]], axis=0)
            pwin = jnp.concatenate([spos[_NH * _S - _BK:], spos[:_GR]],
                                   axis=0)
        else:
            kwin = sk[q0 - _BK:q0 + _GR]
            vwin = sv[q0 - _BK:q0 + _GR]
            pwin = spos[q0 - _BK:q0 + _GR]
        pk = lax.dot_general(pwin, eye, (((0,), (0,)), ((), ())),
                             preferred_element_type=_f32, precision=_PH)

        dots = lax.dot_general(q, kwin, (((1,), (1,)), ((), ())),
                               preferred_element_type=_f32) * (_DH ** -0.5)
        qc = _fiota((_GR, 1), 0) // float(_BK)
        kc = _fiota((1, _KR), 1) // float(_BK) - 1.0
        in_win = jnp.logical_or(kc == qc, kc == qc - 1.0)
        dots = jnp.where(pq < pk, -1e9, dots)
        dots = jnp.where(pq == pk, -1e5, dots)
        dots = jnp.where(in_win, dots, -1e9)
        mx = jnp.max(dots, axis=1, keepdims=True)
        p = jnp.exp(dots - mx)
        ssum = jnp.sum(p, axis=1, keepdims=True)
        logit = mx + jnp.log(ssum)
        o = lax.dot_general(p / ssum, vwin, (((1,), (0,)), ((), ())),
                            preferred_element_type=_f32)
        so_parts.append(jnp.concatenate([o, logit], axis=1))
    so = jnp.concatenate(so_parts, axis=0)
    o_ref[0, 0] = jnp.concatenate(
        [so, jnp.zeros((_NH * _S, _OW - _DH - 1), _f32)], axis=1)


def _attention(st):
    return pl.pallas_call(
        _attn_body,
        grid=(_B, _H),
        in_specs=[pl.BlockSpec((1, 1, _NH * _S, _TW),
                               lambda b, h: (b, h, 0, 0))],
        out_specs=pl.BlockSpec((1, 1, _NH * _S, _OW),
                               lambda b, h: (b, h, 0, 0)),
        out_shape=jax.ShapeDtypeStruct((_B, _H, _NH * _S, _OW), _f32),
        compiler_params=pltpu.CompilerParams(
            dimension_semantics=("parallel", "parallel")),
    )(st)


def _comb_body(u_ref, o_ref):
    u = u_ref[0]
    o0, lg0 = u[0, :, :_DH], u[0, :, _DH:_DH + 1]
    o1, lg1 = u[1, :, :_DH], u[1, :, _DH:_DH + 1]
    mx = jnp.maximum(lg0, lg1)
    lse = mx + jnp.log(jnp.exp(lg0 - mx) + jnp.exp(lg1 - mx))
    o_ref[0] = o0 * jnp.exp(lg0 - lse) + o1 * jnp.exp(lg1 - lse)


def _combine(u):
    return pl.pallas_call(
        _comb_body,
        grid=(_B * _H,),
        in_specs=[pl.BlockSpec((1, _NH, _S, _OW), lambda i: (i, 0, 0, 0))],
        out_specs=pl.BlockSpec((1, _S, _DH), lambda i: (i, 0, 0)),
        out_shape=jax.ShapeDtypeStruct((_B * _H, _S, _DH), _f32),
        compiler_params=pltpu.CompilerParams(
            dimension_semantics=("parallel",)),
    )(u)


def kernel(x, W_emb, b_emb, ln1_s, ln1_b, Wqk, Wv, Wo, ln2_s, ln2_b,
           Wff1, bff1, Wff2, bff2, lnf_s, lnf_b, Wf1, bf1, Wf2, bf2):
    rot3 = jax.random.normal(jax.random.key(42), (_DH, _NH, _NB // 2),
                             dtype=_f32)
    x2 = x.reshape(_ROWS, x.shape[-1])

    def _bucket_ids(h2, s, b, wqk):
        h3 = h2.reshape(_B, _S, _D)
        mu = jnp.mean(h3, axis=-1, keepdims=True)
        var = jnp.var(h3, axis=-1, keepdims=True)
        y = (h3 - mu) / jnp.sqrt(var + 1e-5) * s + b
        qk = (y @ wqk).reshape(_B, _S, _H, _DH).transpose(0, 2, 1, 3)
        rotated = jnp.einsum('bhsd,dnr->bhnsr', qk, rot3)
        bk = jnp.argmax(jnp.concatenate([rotated, -rotated], axis=-1),
                        axis=-1)
        return bk.transpose(0, 1, 3, 2).astype(_f32)

    poscol = (jnp.arange(_N1, dtype=jnp.int32) % _S).astype(_f32)[:, None]
    tpad = jnp.zeros((_N1, _TW - 2 * _DH - 1), _f32)
    bh_arange = jnp.arange(_B * _H, dtype=jnp.int32)
    row_base2 = (jnp.arange(_N2, dtype=jnp.int32) // _S) * _S

    h = _row_grid_call(_embed_body,
                       [(x2, True), (W_emb, False),
                        (b_emb.reshape(1, _D), False)])
    for l in range(Wqk.shape[0]):
        qkh, vh = _row_grid_call(
            _ln_qkv_body,
            [(h, True), (ln1_s[l].reshape(1, _D), False),
             (ln1_b[l].reshape(1, _D), False),
             (Wqk[l], False), (Wv[l], False)],
            n_out=2)
        bks = _bucket_ids(h, ln1_s[l], ln1_b[l], Wqk[l])
        dest = _prep(bks)

        qk_r = qkh.reshape(_B, _S, _H, _DH).transpose(0, 2, 1, 3)
        v_r = vh.reshape(_B, _S, _H, _DH).transpose(0, 2, 1, 3)
        table = jnp.concatenate(
            [qk_r.reshape(_N1, _DH), v_r.reshape(_N1, _DH), poscol, tpad],
            axis=1)

        di = dest.astype(jnp.int32).reshape(_B * _H, _S, _NH)
        idx1 = jnp.concatenate(
            [(((bh_arange * _NH + r) * _S)[:, None] + di[:, :, r]).reshape(-1)
             for r in range(_NH)])
        st = _sc_scatter(table, idx1, _N2, _NH)

        so = _attention(st.reshape(_B, _H, _NH * _S, _TW))

        idx2 = row_base2 + st[:, 2 * _DH].astype(jnp.int32)
        uns = _sc_scatter(so.reshape(_N2, _OW), idx2, _N2, 1)

        a = _combine(uns.reshape(_B * _H, _NH, _S, _OW))
        a2 = a.reshape(_B, _H, _S, _DH).transpose(0, 2, 1, 3).reshape(
            _ROWS, _D)
        h = _row_grid_call(
            _resid_wo_body,
            [(h, True), (a2, True), (Wo[l], False)])
        h = _row_grid_call(
            _ff_body,
            [(h, True), (ln2_s[l].reshape(1, _D), False),
             (ln2_b[l].reshape(1, _D), False),
             (Wff1[l], False), (bff1[l].reshape(1, 4 * _D), False),
             (Wff2[l], False), (bff2[l].reshape(1, _D), False)])

    last = h.reshape(_B, _S, _D)[:, -1, :]
    out = pl.pallas_call(
        _head_body,
        out_shape=jax.ShapeDtypeStruct((_B, Wf2.shape[1]), _f32),
    )(last, lnf_s.reshape(1, _D), lnf_b.reshape(1, _D),
      Wf1, bf1.reshape(1, -1), Wf2, bf2.reshape(1, -1))
    return out


# speedup vs baseline: 5.9429x; 1.0407x over previous
import functools
import jax
import jax.numpy as jnp
from jax import lax
from jax.experimental import pallas as pl
from jax.experimental.pallas import tpu as pltpu
from jax.experimental.pallas import tpu_sc as plsc

_B = 2
_S = 2048
_D = 768
_H = 12
_DH = 64
_NH = 2
_BK = 64
_NB = _S // _BK
_NC = _NH * _NB
_ROWS = _B * _S
_RB = 512
_G = 4
_GR = _G * _BK
_KR = (_G + 1) * _BK
_N1 = _B * _H * _S
_N2 = _B * _H * _NH * _S
_TW = 256
_OW = 128
_NWORK = 32
_SCH = 128

_f32 = jnp.float32
_PH = jax.lax.Precision.HIGHEST


def _fiota(shape, dim):
    return lax.broadcasted_iota(jnp.int32, shape, dim).astype(_f32)


def _ln(x, s, b):
    mu = jnp.mean(x, axis=-1, keepdims=True)
    xc = x - mu
    var = jnp.mean(xc * xc, axis=-1, keepdims=True)
    return xc / jnp.sqrt(var + 1e-5) * s + b


def _embed_body(x_ref, w_ref, b_ref, o_ref):
    o_ref[...] = jnp.dot(x_ref[...], w_ref[...],
                         preferred_element_type=_f32) + b_ref[...]


def _ln_qkv_body(h_ref, s_ref, b_ref, wqk_ref, wv_ref, qk_ref, v_ref):
    y = _ln(h_ref[...], s_ref[...], b_ref[...])
    qk_ref[...] = jnp.dot(y, wqk_ref[...], preferred_element_type=_f32)
    v_ref[...] = jnp.dot(y, wv_ref[...], preferred_element_type=_f32)


def _resid_wo_body(h_ref, a_ref, wo_ref, o_ref):
    o_ref[...] = h_ref[...] + jnp.dot(a_ref[...], wo_ref[...],
                                      preferred_element_type=_f32)


def _ff_body(h_ref, s_ref, b_ref, w1_ref, b1_ref, w2_ref, b2_ref, o_ref):
    h = h_ref[...]
    y = _ln(h, s_ref[...], b_ref[...])
    u = jnp.maximum(jnp.dot(y, w1_ref[...], preferred_element_type=_f32)
                    + b1_ref[...], 0.0)
    o_ref[...] = h + jnp.dot(u, w2_ref[...],
                             preferred_element_type=_f32) + b2_ref[...]


def _head_body(x_ref, s_ref, b_ref, w1_ref, b1_ref, w2_ref, b2_ref, o_ref):
    y = _ln(x_ref[...], s_ref[...], b_ref[...])
    u = jnp.maximum(jnp.dot(y, w1_ref[...], preferred_element_type=_f32)
                    + b1_ref[...], 0.0)
    o_ref[...] = jnp.dot(u, w2_ref[...],
                         preferred_element_type=_f32) + b2_ref[...]


def _row_grid_call(body, ins, n_out=1, out_cols=_D):
    specs = []
    args = []
    for a, blocked in ins:
        args.append(a)
        if blocked:
            specs.append(pl.BlockSpec((_RB, a.shape[1]), lambda i: (i, 0)))
        else:
            specs.append(pl.BlockSpec(a.shape,
                                      lambda i, nd=a.ndim: (0,) * nd))
    out_shape = [jax.ShapeDtypeStruct((_ROWS, out_cols), _f32)
                 for _ in range(n_out)]
    out_specs = [pl.BlockSpec((_RB, out_cols), lambda i: (i, 0))
                 for _ in range(n_out)]
    if n_out == 1:
        out_shape, out_specs = out_shape[0], out_specs[0]
    return pl.pallas_call(
        body,
        grid=(_ROWS // _RB,),
        in_specs=specs,
        out_specs=out_specs,
        out_shape=out_shape,
        compiler_params=pltpu.CompilerParams(
            dimension_semantics=("parallel",)),
    )(*args)


def _prep_body(bk_ref, dest_ref):
    bks = bk_ref[0, 0]

    ii = _fiota((128, 128), 0)
    jj = _fiota((128, 128), 1)
    tril = (jj < ii).astype(_f32)
    bi = _fiota((_NB, _NB), 0)
    bj = _fiota((_NB, _NB), 1)
    ustri = (bi < bj).astype(_f32)
    lane_nb = _fiota((1, _NB), 1)

    for r in range(_NH):
        bucket = bks[:, r:r + 1]
        onehot = (bucket == lane_nb).astype(_f32)
        hist = jnp.sum(onehot, axis=0, keepdims=True)
        offs = jnp.dot(hist, ustri,
                       preferred_element_type=_f32, precision=_PH)
        carry = jnp.zeros((1, _NB), _f32)
        dest_blocks = []
        for blk in range(_S // 128):
            rows = onehot[blk * 128:(blk + 1) * 128]
            pref = jnp.dot(tril, rows,
                           preferred_element_type=_f32, precision=_PH) + carry
            dest_blocks.append(
                jnp.sum((pref + offs) * rows, axis=1, keepdims=True))
            carry = carry + jnp.sum(rows, axis=0, keepdims=True)
        dest_ref[0, 0, :, r:r + 1] = jnp.concatenate(dest_blocks, axis=0)


def _prep(bks):
    return pl.pallas_call(
        _prep_body,
        grid=(_B, _H),
        in_specs=[pl.BlockSpec((1, 1, _S, _NH), lambda b, h: (b, h, 0, 0))],
        out_specs=pl.BlockSpec((1, 1, _S, _NH), lambda b, h: (b, h, 0, 0)),
        out_shape=jax.ShapeDtypeStruct((_B, _H, _S, _NH), _f32),
        compiler_params=pltpu.CompilerParams(
            dimension_semantics=("parallel", "parallel")),
    )(bks)


def _sc_scatter(table, idx, out_rows, rounds):
    n, w = table.shape
    per_w = n // _NWORK
    n_ch = per_w // _SCH
    mesh = plsc.VectorSubcoreMesh(core_axis_name="c", subcore_axis_name="s")

    nb = 2

    @functools.partial(
        pl.kernel, mesh=mesh,
        out_type=jax.ShapeDtypeStruct((out_rows, w), _f32),
        scratch_types=[
            pltpu.VMEM((rounds * nb, _SCH), jnp.int32),
            pltpu.VMEM((nb, _SCH, w), _f32),
            pltpu.SemaphoreType.DMA,
            pltpu.SemaphoreType.DMA,
        ],
    )
    def k(table_hbm, idx_hbm, out_hbm, idx_v, rows_v, lsem, ssem):
        wid = lax.axis_index("s") * 2 + lax.axis_index("c")

        def issue_loads(ch):
            buf = ch % nb
            base = wid * per_w + ch * _SCH
            objs = [pltpu.async_copy(table_hbm.at[pl.ds(base, _SCH)],
                                     rows_v.at[buf], lsem)]
            for r in range(rounds):
                objs.append(pltpu.async_copy(
                    idx_hbm.at[pl.ds(r * n + base, _SCH)],
                    idx_v.at[r * nb + buf], lsem))
            return objs

        loads = {0: issue_loads(0)}
        scats = {}
        for ch in range(n_ch):
            buf = ch % nb
            for o in loads.pop(ch):
                o.wait()
            if ch >= 1:
                for o in scats.pop(ch - 1):
                    o.wait()
            if ch + 1 < n_ch:
                loads[ch + 1] = issue_loads(ch + 1)
            scats[ch] = [pltpu.async_copy(rows_v.at[buf],
                                          out_hbm.at[idx_v.at[r * nb + buf]],
                                          ssem)
                         for r in range(rounds)]
        for o in scats.pop(n_ch - 1):
            o.wait()

    return k(table, idx)


def _attn_body(st_ref, o_ref):
    blk = st_ref[0, 0]
    sqk = blk[:, :_DH]
    sv = blk[:, _DH:2 * _DH]
    spos = blk[:, 2 * _DH:2 * _DH + 1]
    nrm = jnp.sqrt(jnp.sum(sqk * sqk, axis=1, keepdims=True))
    sk = sqk / (nrm + 1e-6)

    ei = _fiota((_KR, _KR), 0)
    ej = _fiota((_KR, _KR), 1)
    eye = (ei == ej).astype(_f32)

    n_groups = _NC // _G
    so_parts = []
    for gidx in range(n_groups):
        q0 = gidx * _GR
        q = sqk[q0:q0 + _GR]
        pq = spos[q0:q0 + _GR]
        if gidx == 0:
            kwin = jnp.concatenate([sk[_NH * _S - _BK:], sk[:_GR]], axis=0)
            vwin = jnp.concatenate([sv[_NH * _S - _BK:], sv[:_GR]], axis=0)
            pwin = jnp.concatenate([spos[_NH * _S - _BK:], spos[:_GR]],
                                   axis=0)
        else:
            kwin = sk[q0 - _BK:q0 + _GR]
            vwin = sv[q0 - _BK:q0 + _GR]
            pwin = spos[q0 - _BK:q0 + _GR]
        pk = lax.dot_general(pwin, eye, (((0,), (0,)), ((), ())),
                             preferred_element_type=_f32, precision=_PH)

        dots = lax.dot_general(q, kwin, (((1,), (1,)), ((), ())),
                               preferred_element_type=_f32) * (_DH ** -0.5)
        qc = _fiota((_GR, 1), 0) // float(_BK)
        kc = _fiota((1, _KR), 1) // float(_BK) - 1.0
        in_win = jnp.logical_or(kc == qc, kc == qc - 1.0)
        dots = jnp.where(pq < pk, -1e9, dots)
        dots = jnp.where(pq == pk, -1e5, dots)
        dots = jnp.where(in_win, dots, -1e9)
        mx = jnp.max(dots, axis=1, keepdims=True)
        p = jnp.exp(dots - mx)
        ssum = jnp.sum(p, axis=1, keepdims=True)
        logit = mx + jnp.log(ssum)
        o = lax.dot_general(p / ssum, vwin, (((1,), (0,)), ((), ())),
                            preferred_element_type=_f32)
        so_parts.append(jnp.concatenate([o, logit], axis=1))
    so = jnp.concatenate(so_parts, axis=0)
    o_ref[0, 0] = jnp.concatenate(
        [so, jnp.zeros((_NH * _S, _OW - _DH - 1), _f32)], axis=1)


def _attention(st):
    return pl.pallas_call(
        _attn_body,
        grid=(_B, _H),
        in_specs=[pl.BlockSpec((1, 1, _NH * _S, _TW),
                               lambda b, h: (b, h, 0, 0))],
        out_specs=pl.BlockSpec((1, 1, _NH * _S, _OW),
                               lambda b, h: (b, h, 0, 0)),
        out_shape=jax.ShapeDtypeStruct((_B, _H, _NH * _S, _OW), _f32),
        compiler_params=pltpu.CompilerParams(
            dimension_semantics=("parallel", "parallel")),
    )(st)


def _comb_body(u_ref, o_ref):
    u = u_ref[0]
    o0, lg0 = u[0, :, :_DH], u[0, :, _DH:_DH + 1]
    o1, lg1 = u[1, :, :_DH], u[1, :, _DH:_DH + 1]
    mx = jnp.maximum(lg0, lg1)
    lse = mx + jnp.log(jnp.exp(lg0 - mx) + jnp.exp(lg1 - mx))
    o_ref[0] = o0 * jnp.exp(lg0 - lse) + o1 * jnp.exp(lg1 - lse)


def _combine(u):
    return pl.pallas_call(
        _comb_body,
        grid=(_B * _H,),
        in_specs=[pl.BlockSpec((1, _NH, _S, _OW), lambda i: (i, 0, 0, 0))],
        out_specs=pl.BlockSpec((1, _S, _DH), lambda i: (i, 0, 0)),
        out_shape=jax.ShapeDtypeStruct((_B * _H, _S, _DH), _f32),
        compiler_params=pltpu.CompilerParams(
            dimension_semantics=("parallel",)),
    )(u)


def kernel(x, W_emb, b_emb, ln1_s, ln1_b, Wqk, Wv, Wo, ln2_s, ln2_b,
           Wff1, bff1, Wff2, bff2, lnf_s, lnf_b, Wf1, bf1, Wf2, bf2):
    rot3 = jax.random.normal(jax.random.key(42), (_DH, _NH, _NB // 2),
                             dtype=_f32)
    x2 = x.reshape(_ROWS, x.shape[-1])

    def _bucket_ids(h2, s, b, wqk):
        h3 = h2.reshape(_B, _S, _D)
        mu = jnp.mean(h3, axis=-1, keepdims=True)
        var = jnp.var(h3, axis=-1, keepdims=True)
        y = (h3 - mu) / jnp.sqrt(var + 1e-5) * s + b
        qk = (y @ wqk).reshape(_B, _S, _H, _DH).transpose(0, 2, 1, 3)
        rotated = jnp.einsum('bhsd,dnr->bhnsr', qk, rot3)
        bk = jnp.argmax(jnp.concatenate([rotated, -rotated], axis=-1),
                        axis=-1)
        return bk.transpose(0, 1, 3, 2).astype(_f32)

    poscol = (jnp.arange(_N1, dtype=jnp.int32) % _S).astype(_f32)[:, None]
    tpad = jnp.zeros((_N1, _TW - 2 * _DH - 1), _f32)
    bh_arange = jnp.arange(_B * _H, dtype=jnp.int32)
    row_base2 = (jnp.arange(_N2, dtype=jnp.int32) // _S) * _S

    h = _row_grid_call(_embed_body,
                       [(x2, True), (W_emb, False),
                        (b_emb.reshape(1, _D), False)])
    for l in range(Wqk.shape[0]):
        qkh, vh = _row_grid_call(
            _ln_qkv_body,
            [(h, True), (ln1_s[l].reshape(1, _D), False),
             (ln1_b[l].reshape(1, _D), False),
             (Wqk[l], False), (Wv[l], False)],
            n_out=2)
        bks = _bucket_ids(h, ln1_s[l], ln1_b[l], Wqk[l])
        dest = _prep(bks)

        qk_r = qkh.reshape(_B, _S, _H, _DH).transpose(0, 2, 1, 3)
        v_r = vh.reshape(_B, _S, _H, _DH).transpose(0, 2, 1, 3)
        table = jnp.concatenate(
            [qk_r.reshape(_N1, _DH), v_r.reshape(_N1, _DH), poscol, tpad],
            axis=1)

        di = dest.astype(jnp.int32).reshape(_B * _H, _S, _NH)
        idx1 = jnp.concatenate(
            [(((bh_arange * _NH + r) * _S)[:, None] + di[:, :, r]).reshape(-1)
             for r in range(_NH)])
        st = _sc_scatter(table, idx1, _N2, _NH)

        so = _attention(st.reshape(_B, _H, _NH * _S, _TW))

        idx2 = row_base2 + st[:, 2 * _DH].astype(jnp.int32)
        uns = _sc_scatter(so.reshape(_N2, _OW), idx2, _N2, 1)

        a = _combine(uns.reshape(_B * _H, _NH, _S, _OW))
        a2 = a.reshape(_B, _H, _S, _DH).transpose(0, 2, 1, 3).reshape(
            _ROWS, _D)
        h = _row_grid_call(
            _resid_wo_body,
            [(h, True), (a2, True), (Wo[l], False)])
        h = _row_grid_call(
            _ff_body,
            [(h, True), (ln2_s[l].reshape(1, _D), False),
             (ln2_b[l].reshape(1, _D), False),
             (Wff1[l], False), (bff1[l].reshape(1, 4 * _D), False),
             (Wff2[l], False), (bff2[l].reshape(1, _D), False)])

    last = h.reshape(_B, _S, _D)[:, -1, :]
    out = pl.pallas_call(
        _head_body,
        out_shape=jax.ShapeDtypeStruct((_B, Wf2.shape[1]), _f32),
    )(last, lnf_s.reshape(1, _D), lnf_b.reshape(1, _D),
      Wf1, bf1.reshape(1, -1), Wf2, bf2.reshape(1, -1))
    return out


# speedup vs baseline: 7.2039x; 1.2122x over previous
import functools
import jax
import jax.numpy as jnp
from jax import lax
from jax.experimental import pallas as pl
from jax.experimental.pallas import tpu as pltpu
from jax.experimental.pallas import tpu_sc as plsc

_B = 2
_S = 2048
_D = 768
_H = 12
_DH = 64
_NH = 2
_BK = 64
_NB = _S // _BK
_NC = _NH * _NB
_ROWS = _B * _S
_RB = 512
_G = 4
_GR = _G * _BK
_KR = (_G + 1) * _BK
_N1 = _B * _H * _S
_N2 = _B * _H * _NH * _S
_TW = 256
_OW = 128
_NWORK = 32
_SCH = 128

_f32 = jnp.float32
_PH = jax.lax.Precision.HIGHEST


def _fiota(shape, dim):
    return lax.broadcasted_iota(jnp.int32, shape, dim).astype(_f32)


def _ln(x, s, b):
    mu = jnp.mean(x, axis=-1, keepdims=True)
    xc = x - mu
    var = jnp.mean(xc * xc, axis=-1, keepdims=True)
    return xc / jnp.sqrt(var + 1e-5) * s + b


def _embed_body(x_ref, w_ref, b_ref, o_ref):
    o_ref[...] = jnp.dot(x_ref[...], w_ref[...],
                         preferred_element_type=_f32) + b_ref[...]


_S2 = _S // 2


def _qkvt_body(h_ref, s_ref, b_ref, wqk_ref, wv_ref, o_ref):
    y = _ln(h_ref[0, 0], s_ref[...], b_ref[...])
    qk = jnp.dot(y, wqk_ref[...], preferred_element_type=_f32)
    v = jnp.dot(y, wv_ref[...], preferred_element_type=_f32)
    pos = _fiota((_S2, 1), 0) + pl.program_id(1).astype(_f32) * _S2
    pad = jnp.zeros((_S2, _TW - 2 * _DH - 1), _f32)
    for hh in range(_H):
        o_ref[0, hh] = jnp.concatenate(
            [qk[:, hh * _DH:(hh + 1) * _DH],
             v[:, hh * _DH:(hh + 1) * _DH], pos, pad], axis=1)


def _qkv_table(h3, s, b, wqk, wv):
    return pl.pallas_call(
        _qkvt_body,
        grid=(_B, 2),
        in_specs=[
            pl.BlockSpec((1, 1, _S2, _D), lambda bb, k: (bb, k, 0, 0)),
            pl.BlockSpec((1, _D), lambda bb, k: (0, 0)),
            pl.BlockSpec((1, _D), lambda bb, k: (0, 0)),
            pl.BlockSpec((_D, _D), lambda bb, k: (0, 0)),
            pl.BlockSpec((_D, _D), lambda bb, k: (0, 0)),
        ],
        out_specs=pl.BlockSpec((1, _H, _S2, _TW),
                               lambda bb, k: (bb, 0, k, 0)),
        out_shape=jax.ShapeDtypeStruct((_B, _H, _S, _TW), _f32),
        compiler_params=pltpu.CompilerParams(
            dimension_semantics=("parallel", "parallel")),
    )(h3.reshape(_B, 2, _S2, _D), s, b, wqk, wv)


def _resid_wo_body(h_ref, a_ref, wo_ref, o_ref):
    acc = h_ref[...]
    for hh in range(_H):
        acc = acc + jnp.dot(a_ref[0, hh], wo_ref[hh * _DH:(hh + 1) * _DH],
                            preferred_element_type=_f32)
    o_ref[...] = acc


def _resid_wo(h, a4, wo):
    nrb = _ROWS // _RB
    per_b = _S // _RB
    return pl.pallas_call(
        _resid_wo_body,
        grid=(nrb,),
        in_specs=[
            pl.BlockSpec((_RB, _D), lambda i: (i, 0)),
            pl.BlockSpec((1, _H, _RB, _DH),
                         lambda i, pb=per_b: (i // pb, 0, i % pb, 0)),
            pl.BlockSpec((_D, _D), lambda i: (0, 0)),
        ],
        out_specs=pl.BlockSpec((_RB, _D), lambda i: (i, 0)),
        out_shape=jax.ShapeDtypeStruct((_ROWS, _D), _f32),
        compiler_params=pltpu.CompilerParams(
            dimension_semantics=("parallel",)),
    )(h, a4, wo)


def _ff_body(h_ref, s_ref, b_ref, w1_ref, b1_ref, w2_ref, b2_ref, o_ref):
    h = h_ref[...]
    y = _ln(h, s_ref[...], b_ref[...])
    u = jnp.maximum(jnp.dot(y, w1_ref[...], preferred_element_type=_f32)
                    + b1_ref[...], 0.0)
    o_ref[...] = h + jnp.dot(u, w2_ref[...],
                             preferred_element_type=_f32) + b2_ref[...]


def _head_body(x_ref, s_ref, b_ref, w1_ref, b1_ref, w2_ref, b2_ref, o_ref):
    y = _ln(x_ref[...], s_ref[...], b_ref[...])
    u = jnp.maximum(jnp.dot(y, w1_ref[...], preferred_element_type=_f32)
                    + b1_ref[...], 0.0)
    o_ref[...] = jnp.dot(u, w2_ref[...],
                         preferred_element_type=_f32) + b2_ref[...]


def _row_grid_call(body, ins, n_out=1, out_cols=_D):
    specs = []
    args = []
    for a, blocked in ins:
        args.append(a)
        if blocked:
            specs.append(pl.BlockSpec((_RB, a.shape[1]), lambda i: (i, 0)))
        else:
            specs.append(pl.BlockSpec(a.shape,
                                      lambda i, nd=a.ndim: (0,) * nd))
    out_shape = [jax.ShapeDtypeStruct((_ROWS, out_cols), _f32)
                 for _ in range(n_out)]
    out_specs = [pl.BlockSpec((_RB, out_cols), lambda i: (i, 0))
                 for _ in range(n_out)]
    if n_out == 1:
        out_shape, out_specs = out_shape[0], out_specs[0]
    return pl.pallas_call(
        body,
        grid=(_ROWS // _RB,),
        in_specs=specs,
        out_specs=out_specs,
        out_shape=out_shape,
        compiler_params=pltpu.CompilerParams(
            dimension_semantics=("parallel",)),
    )(*args)


def _prep_body(bk_ref, dest_ref):
    bks = bk_ref[0, 0]

    ii = _fiota((128, 128), 0)
    jj = _fiota((128, 128), 1)
    tril = (jj < ii).astype(_f32)
    bi = _fiota((_NB, _NB), 0)
    bj = _fiota((_NB, _NB), 1)
    ustri = (bi < bj).astype(_f32)
    lane_nb = _fiota((1, _NB), 1)

    for r in range(_NH):
        bucket = bks[:, r:r + 1]
        onehot = (bucket == lane_nb).astype(_f32)
        hist = jnp.sum(onehot, axis=0, keepdims=True)
        offs = jnp.dot(hist, ustri,
                       preferred_element_type=_f32, precision=_PH)
        carry = jnp.zeros((1, _NB), _f32)
        dest_blocks = []
        for blk in range(_S // 128):
            rows = onehot[blk * 128:(blk + 1) * 128]
            pref = jnp.dot(tril, rows,
                           preferred_element_type=_f32, precision=_PH) + carry
            dest_blocks.append(
                jnp.sum((pref + offs) * rows, axis=1, keepdims=True))
            carry = carry + jnp.sum(rows, axis=0, keepdims=True)
        dest_ref[0, 0, :, r:r + 1] = jnp.concatenate(dest_blocks, axis=0)


def _prep(bks):
    return pl.pallas_call(
        _prep_body,
        grid=(_B, _H),
        in_specs=[pl.BlockSpec((1, 1, _S, _NH), lambda b, h: (b, h, 0, 0))],
        out_specs=pl.BlockSpec((1, 1, _S, _NH), lambda b, h: (b, h, 0, 0)),
        out_shape=jax.ShapeDtypeStruct((_B, _H, _S, _NH), _f32),
        compiler_params=pltpu.CompilerParams(
            dimension_semantics=("parallel", "parallel")),
    )(bks)


def _sc_scatter(table, idx, out_rows, rounds):
    n, w = table.shape
    per_w = n // _NWORK
    n_ch = per_w // _SCH
    mesh = plsc.VectorSubcoreMesh(core_axis_name="c", subcore_axis_name="s")

    nb = 2

    @functools.partial(
        pl.kernel, mesh=mesh,
        out_type=jax.ShapeDtypeStruct((out_rows, w), _f32),
        scratch_types=[
            pltpu.VMEM((rounds * nb, _SCH), jnp.int32),
            pltpu.VMEM((nb, _SCH, w), _f32),
            pltpu.SemaphoreType.DMA,
            pltpu.SemaphoreType.DMA,
        ],
    )
    def k(table_hbm, idx_hbm, out_hbm, idx_v, rows_v, lsem, ssem):
        wid = lax.axis_index("s") * 2 + lax.axis_index("c")

        def issue_loads(ch):
            buf = ch % nb
            base = wid * per_w + ch * _SCH
            objs = [pltpu.async_copy(table_hbm.at[pl.ds(base, _SCH)],
                                     rows_v.at[buf], lsem)]
            for r in range(rounds):
                objs.append(pltpu.async_copy(
                    idx_hbm.at[pl.ds(r * n + base, _SCH)],
                    idx_v.at[r * nb + buf], lsem))
            return objs

        loads = {0: issue_loads(0)}
        scats = {}
        for ch in range(n_ch):
            buf = ch % nb
            for o in loads.pop(ch):
                o.wait()
            if ch >= 1:
                for o in scats.pop(ch - 1):
                    o.wait()
            if ch + 1 < n_ch:
                loads[ch + 1] = issue_loads(ch + 1)
            scats[ch] = [pltpu.async_copy(rows_v.at[buf],
                                          out_hbm.at[idx_v.at[r * nb + buf]],
                                          ssem)
                         for r in range(rounds)]
        for o in scats.pop(n_ch - 1):
            o.wait()

    return k(table, idx)


def _attn_body(st_ref, o_ref):
    blk = st_ref[0, 0]
    sqk = blk[:, :_DH]
    sv = blk[:, _DH:2 * _DH]
    spos = blk[:, 2 * _DH:2 * _DH + 1]
    nrm = jnp.sqrt(jnp.sum(sqk * sqk, axis=1, keepdims=True))
    sk = sqk / (nrm + 1e-6)

    ei = _fiota((_KR, _KR), 0)
    ej = _fiota((_KR, _KR), 1)
    eye = (ei == ej).astype(_f32)

    n_groups = _NC // _G
    so_parts = []
    for gidx in range(n_groups):
        q0 = gidx * _GR
        q = sqk[q0:q0 + _GR]
        pq = spos[q0:q0 + _GR]
        if gidx == 0:
            kwin = jnp.concatenate([sk[_NH * _S - _BK:], sk[:_GR]], axis=0)
            vwin = jnp.concatenate([sv[_NH * _S - _BK:], sv[:_GR]], axis=0)
            pwin = jnp.concatenate([spos[_NH * _S - _BK:], spos[:_GR]],
                                   axis=0)
        else:
            kwin = sk[q0 - _BK:q0 + _GR]
            vwin = sv[q0 - _BK:q0 + _GR]
            pwin = spos[q0 - _BK:q0 + _GR]
        pk = lax.dot_general(pwin, eye, (((0,), (0,)), ((), ())),
                             preferred_element_type=_f32, precision=_PH)

        dots = lax.dot_general(q, kwin, (((1,), (1,)), ((), ())),
                               preferred_element_type=_f32) * (_DH ** -0.5)
        qc = _fiota((_GR, 1), 0) // float(_BK)
        kc = _fiota((1, _KR), 1) // float(_BK) - 1.0
        in_win = jnp.logical_or(kc == qc, kc == qc - 1.0)
        dots = jnp.where(pq < pk, -1e9, dots)
        dots = jnp.where(pq == pk, -1e5, dots)
        dots = jnp.where(in_win, dots, -1e9)
        mx = jnp.max(dots, axis=1, keepdims=True)
        p = jnp.exp(dots - mx)
        ssum = jnp.sum(p, axis=1, keepdims=True)
        logit = mx + jnp.log(ssum)
        o = lax.dot_general(p / ssum, vwin, (((1,), (0,)), ((), ())),
                            preferred_element_type=_f32)
        so_parts.append(jnp.concatenate([o, logit], axis=1))
    so = jnp.concatenate(so_parts, axis=0)
    o_ref[0, 0] = jnp.concatenate(
        [so, jnp.zeros((_NH * _S, _OW - _DH - 1), _f32)], axis=1)


def _attention(st):
    return pl.pallas_call(
        _attn_body,
        grid=(_B, _H),
        in_specs=[pl.BlockSpec((1, 1, _NH * _S, _TW),
                               lambda b, h: (b, h, 0, 0))],
        out_specs=pl.BlockSpec((1, 1, _NH * _S, _OW),
                               lambda b, h: (b, h, 0, 0)),
        out_shape=jax.ShapeDtypeStruct((_B, _H, _NH * _S, _OW), _f32),
        compiler_params=pltpu.CompilerParams(
            dimension_semantics=("parallel", "parallel")),
    )(st)


def _comb_body(u_ref, o_ref):
    u = u_ref[0]
    o0, lg0 = u[0, :, :_DH], u[0, :, _DH:_DH + 1]
    o1, lg1 = u[1, :, :_DH], u[1, :, _DH:_DH + 1]
    mx = jnp.maximum(lg0, lg1)
    lse = mx + jnp.log(jnp.exp(lg0 - mx) + jnp.exp(lg1 - mx))
    o_ref[0] = o0 * jnp.exp(lg0 - lse) + o1 * jnp.exp(lg1 - lse)


def _combine(u):
    return pl.pallas_call(
        _comb_body,
        grid=(_B * _H,),
        in_specs=[pl.BlockSpec((1, _NH, _S, _OW), lambda i: (i, 0, 0, 0))],
        out_specs=pl.BlockSpec((1, _S, _DH), lambda i: (i, 0, 0)),
        out_shape=jax.ShapeDtypeStruct((_B * _H, _S, _DH), _f32),
        compiler_params=pltpu.CompilerParams(
            dimension_semantics=("parallel",)),
    )(u)


def kernel(x, W_emb, b_emb, ln1_s, ln1_b, Wqk, Wv, Wo, ln2_s, ln2_b,
           Wff1, bff1, Wff2, bff2, lnf_s, lnf_b, Wf1, bf1, Wf2, bf2):
    rot3 = jax.random.normal(jax.random.key(42), (_DH, _NH, _NB // 2),
                             dtype=_f32)
    x2 = x.reshape(_ROWS, x.shape[-1])

    def _bucket_ids(h2, s, b, wqk):
        h3 = h2.reshape(_B, _S, _D)
        mu = jnp.mean(h3, axis=-1, keepdims=True)
        var = jnp.var(h3, axis=-1, keepdims=True)
        y = (h3 - mu) / jnp.sqrt(var + 1e-5) * s + b
        qk = (y @ wqk).reshape(_B, _S, _H, _DH).transpose(0, 2, 1, 3)
        rotated = jnp.einsum('bhsd,dnr->bhnsr', qk, rot3)
        bk = jnp.argmax(jnp.concatenate([rotated, -rotated], axis=-1),
                        axis=-1)
        return bk.transpose(0, 1, 3, 2).astype(_f32)

    bh_arange = jnp.arange(_B * _H, dtype=jnp.int32)
    row_base2 = (jnp.arange(_N2, dtype=jnp.int32) // _S) * _S

    h = _row_grid_call(_embed_body,
                       [(x2, True), (W_emb, False),
                        (b_emb.reshape(1, _D), False)])
    for l in range(Wqk.shape[0]):
        table4 = _qkv_table(h.reshape(_B, _S, _D),
                            ln1_s[l].reshape(1, _D), ln1_b[l].reshape(1, _D),
                            Wqk[l], Wv[l])
        table = table4.reshape(_N1, _TW)
        bks = _bucket_ids(h, ln1_s[l], ln1_b[l], Wqk[l])
        dest = _prep(bks)

        di = dest.astype(jnp.int32).reshape(_B * _H, _S, _NH)
        idx1 = jnp.concatenate(
            [(((bh_arange * _NH + r) * _S)[:, None] + di[:, :, r]).reshape(-1)
             for r in range(_NH)])
        st = _sc_scatter(table, idx1, _N2, _NH)

        so = _attention(st.reshape(_B, _H, _NH * _S, _TW))

        idx2 = row_base2 + st[:, 2 * _DH].astype(jnp.int32)
        uns = _sc_scatter(so.reshape(_N2, _OW), idx2, _N2, 1)

        a = _combine(uns.reshape(_B * _H, _NH, _S, _OW))
        h = _resid_wo(h, a.reshape(_B, _H, _S, _DH), Wo[l])
        h = _row_grid_call(
            _ff_body,
            [(h, True), (ln2_s[l].reshape(1, _D), False),
             (ln2_b[l].reshape(1, _D), False),
             (Wff1[l], False), (bff1[l].reshape(1, 4 * _D), False),
             (Wff2[l], False), (bff2[l].reshape(1, _D), False)])

    last = h.reshape(_B, _S, _D)[:, -1, :]
    out = pl.pallas_call(
        _head_body,
        out_shape=jax.ShapeDtypeStruct((_B, Wf2.shape[1]), _f32),
    )(last, lnf_s.reshape(1, _D), lnf_b.reshape(1, _D),
      Wf1, bf1.reshape(1, -1), Wf2, bf2.reshape(1, -1))
    return out
